# baseline (speedup 1.0000x reference)
import functools

import jax
import jax.numpy as jnp
from jax import lax
from jax.experimental import pallas as pl
from jax.experimental.pallas import tpu as pltpu
from jax.experimental.pallas import tpu_sc as plsc

_N_ITEMS = 1_000_000
_D = 32
_L = 50
_B = 16384
_LANES = 16
_NC = 2
_NS = 16
_NW = _NC * _NS
_BW = _B // _NW
_CH = 512
_NCH = (_BW * _L) // _CH

_BB = 128
_GRID = _B // _BB


def _sc_gather_body(user_ids, pos_ids, neg_ids, pref_ids_flat,
                    user_emb, item_emb,
                    out_pref, out_u, out_p, out_n,
                    idbuf, uid_v, pid_v, nid_v,
                    rows0, rows1, urows, prows, nrows,
                    sem_in0, sem_in1, sem_out0, sem_out1,
                    sem_upn_in, sem_upn_out):
    wid = lax.axis_index("s") * _NC + lax.axis_index("c")
    base_b = wid * _BW
    base_r = base_b * _L

    pltpu.sync_copy(pref_ids_flat.at[pl.ds(base_r, _BW * _L)], idbuf)
    pltpu.sync_copy(user_ids.at[pl.ds(base_b, _BW)], uid_v)
    pltpu.sync_copy(pos_ids.at[pl.ds(base_b, _BW)], pid_v)
    pltpu.sync_copy(neg_ids.at[pl.ds(base_b, _BW)], nid_v)

    def clamp_body(i, _):
        off = i * _LANES
        v = idbuf[pl.ds(off, _LANES)]
        idbuf[pl.ds(off, _LANES)] = jnp.minimum(
            v, jnp.full((_LANES,), _N_ITEMS - 1, jnp.int32))
        return 0
    lax.fori_loop(0, (_BW * _L) // _LANES, clamp_body, 0)

    hu = pltpu.async_copy(user_emb.at[uid_v], urows, sem_upn_in)
    hp = pltpu.async_copy(item_emb.at[pid_v], prows, sem_upn_in)
    hn = pltpu.async_copy(item_emb.at[nid_v], nrows, sem_upn_in)
    hu.wait()
    hp.wait()
    hn.wait()
    ou = pltpu.async_copy(urows, out_u.at[pl.ds(base_b, _BW)], sem_upn_out)
    op = pltpu.async_copy(prows, out_p.at[pl.ds(base_b, _BW)], sem_upn_out)
    on = pltpu.async_copy(nrows, out_n.at[pl.ds(base_b, _BW)], sem_upn_out)

    bufs = (rows0, rows1)
    sems_in = (sem_in0, sem_in1)
    sems_out = (sem_out0, sem_out1)
    h_in = [None, None]
    h_out = [None, None]
    h_in[0] = pltpu.async_copy(
        item_emb.at[idbuf.at[pl.ds(0, _CH)]], bufs[0], sems_in[0])
    for c in range(_NCH):
        cur = c % 2
        nxt = 1 - cur
        if c + 1 < _NCH:
            if h_out[nxt] is not None:
                h_out[nxt].wait()
            h_in[nxt] = pltpu.async_copy(
                item_emb.at[idbuf.at[pl.ds((c + 1) * _CH, _CH)]],
                bufs[nxt], sems_in[nxt])
        h_in[cur].wait()
        h_out[cur] = pltpu.async_copy(
            bufs[cur], out_pref.at[pl.ds(base_r + c * _CH, _CH)],
            sems_out[cur])
    h_out[(_NCH - 1) % 2].wait()

    ou.wait()
    op.wait()
    on.wait()


@jax.jit
def _sc_gather(user_ids, pos_ids, neg_ids, pref_ids_flat,
               user_emb, item_emb):
    mesh = plsc.VectorSubcoreMesh(core_axis_name="c", subcore_axis_name="s")
    f = pl.kernel(
        _sc_gather_body,
        compiler_params=pltpu.CompilerParams(use_tc_tiling_on_sc=False),
        out_type=[
            jax.ShapeDtypeStruct((_B * _L, _D), jnp.float32),
            jax.ShapeDtypeStruct((_B, _D), jnp.float32),
            jax.ShapeDtypeStruct((_B, _D), jnp.float32),
            jax.ShapeDtypeStruct((_B, _D), jnp.float32),
        ],
        mesh=mesh,
        scratch_types=[
            pltpu.VMEM((_BW * _L,), jnp.int32),
            pltpu.VMEM((_BW,), jnp.int32),
            pltpu.VMEM((_BW,), jnp.int32),
            pltpu.VMEM((_BW,), jnp.int32),
            pltpu.VMEM((_CH, _D), jnp.float32),
            pltpu.VMEM((_CH, _D), jnp.float32),
            pltpu.VMEM((_BW, _D), jnp.float32),
            pltpu.VMEM((_BW, _D), jnp.float32),
            pltpu.VMEM((_BW, _D), jnp.float32),
            pltpu.SemaphoreType.DMA,
            pltpu.SemaphoreType.DMA,
            pltpu.SemaphoreType.DMA,
            pltpu.SemaphoreType.DMA,
            pltpu.SemaphoreType.DMA,
            pltpu.SemaphoreType.DMA,
        ],
    )
    return f(user_ids, pos_ids, neg_ids, pref_ids_flat, user_emb, item_emb)


def _tc_att_body(pref_ref, u_ref, p_ref, n_ref, ids_ref, npr_ref,
                 dpos_ref, dneg_ref):
    pref = pref_ref[...]
    u = u_ref[...]
    p = p_ref[...]
    n = n_ref[...]
    ids = ids_ref[...]
    npr = npr_ref[...]

    valid = (ids < _N_ITEMS).astype(jnp.float32)
    lpos = lax.broadcasted_iota(jnp.int32, (_BB, _L), 1).astype(jnp.float32)
    lmask = (lpos < (npr + 1.0)).astype(jnp.float32)

    pref_z = pref * valid[:, :, None]

    def att_pool(target):
        w = jnp.sum(pref_z * target[:, None, :], axis=-1)
        e = jnp.exp(w) * lmask
        s = jnp.sum(e, axis=1, keepdims=True)
        att = e / s
        return jnp.sum(pref_z * att[:, :, None], axis=1)

    u_pos = u + att_pool(p)
    u_neg = u + att_pool(n)
    dpos_ref[...] = jnp.sum(jnp.square(u_pos - p), axis=1)
    dneg_ref[...] = jnp.sum(jnp.square(u_neg - n), axis=1)


@jax.jit
def _tc_att(pref_rows, u_rows, p_rows, n_rows, pref_ids, n_prefs_f):
    return pl.pallas_call(
        _tc_att_body,
        grid=(_GRID,),
        in_specs=[
            pl.BlockSpec((_BB, _L, _D), lambda i: (i, 0, 0)),
            pl.BlockSpec((_BB, _D), lambda i: (i, 0)),
            pl.BlockSpec((_BB, _D), lambda i: (i, 0)),
            pl.BlockSpec((_BB, _D), lambda i: (i, 0)),
            pl.BlockSpec((_BB, _L), lambda i: (i, 0)),
            pl.BlockSpec((_BB, 1), lambda i: (i, 0)),
        ],
        out_specs=[
            pl.BlockSpec((_BB,), lambda i: (i,)),
            pl.BlockSpec((_BB,), lambda i: (i,)),
        ],
        out_shape=[
            jax.ShapeDtypeStruct((_B,), jnp.float32),
            jax.ShapeDtypeStruct((_B,), jnp.float32),
        ],
    )(pref_rows, u_rows, p_rows, n_rows, pref_ids, n_prefs_f)


def kernel(user_ids, pos_ids, neg_ids, pref_ids, n_prefs,
           user_embeddings, item_embeddings):
    pref_flat = pref_ids.reshape(-1)
    pref_rows, u_rows, p_rows, n_rows = _sc_gather(
        user_ids, pos_ids, neg_ids, pref_flat,
        user_embeddings, item_embeddings)
    pref_rows = pref_rows.reshape(_B, _L, _D)
    n_prefs_f = n_prefs.astype(jnp.float32).reshape(_B, 1)
    dpos, dneg = _tc_att(pref_rows, u_rows, p_rows, n_rows,
                         pref_ids, n_prefs_f)
    return (dpos, dneg)


# speedup vs baseline: 1.4988x; 1.4988x over previous
import functools

import jax
import jax.numpy as jnp
from jax import lax
from jax.experimental import pallas as pl
from jax.experimental.pallas import tpu as pltpu
from jax.experimental.pallas import tpu_sc as plsc

_N_ITEMS = 1_000_000
_D = 32
_L = 50
_B = 16384
_LANES = 16
_NC = 2
_NS = 16
_NW = _NC * _NS
_BW = _B // _NW
_CH = 512
_NCH = (_BW * _L) // _CH

_BB = 128
_GRID = _B // _BB


def _sc_gather_body(user_ids, pos_ids, neg_ids, pref_ids_flat,
                    user_emb, item_emb,
                    out_pref, out_u, out_p, out_n,
                    idbuf, uid_v, pid_v, nid_v,
                    rows0, rows1, urows, prows, nrows,
                    sem_in0, sem_in1, sem_out0, sem_out1,
                    sem_upn_in, sem_upn_out):
    wid = lax.axis_index("s") * _NC + lax.axis_index("c")
    base_b = wid * _BW
    base_r = base_b * _L

    pltpu.sync_copy(pref_ids_flat.at[pl.ds(base_r, _BW * _L)], idbuf)
    pltpu.sync_copy(user_ids.at[pl.ds(base_b, _BW)], uid_v)
    pltpu.sync_copy(pos_ids.at[pl.ds(base_b, _BW)], pid_v)
    pltpu.sync_copy(neg_ids.at[pl.ds(base_b, _BW)], nid_v)

    def clamp_body(i, _):
        off = i * _LANES
        v = idbuf[pl.ds(off, _LANES)]
        idbuf[pl.ds(off, _LANES)] = jnp.minimum(
            v, jnp.full((_LANES,), _N_ITEMS - 1, jnp.int32))
        return 0
    lax.fori_loop(0, (_BW * _L) // _LANES, clamp_body, 0)

    hu = pltpu.async_copy(user_emb.at[uid_v], urows, sem_upn_in)
    hp = pltpu.async_copy(item_emb.at[pid_v], prows, sem_upn_in)
    hn = pltpu.async_copy(item_emb.at[nid_v], nrows, sem_upn_in)
    hu.wait()
    hp.wait()
    hn.wait()
    ou = pltpu.async_copy(urows, out_u.at[pl.ds(base_b, _BW)], sem_upn_out)
    op = pltpu.async_copy(prows, out_p.at[pl.ds(base_b, _BW)], sem_upn_out)
    on = pltpu.async_copy(nrows, out_n.at[pl.ds(base_b, _BW)], sem_upn_out)

    bufs = (rows0, rows1)
    sems_in = (sem_in0, sem_in1)
    sems_out = (sem_out0, sem_out1)
    h_in = [None, None]
    h_out = [None, None]
    h_in[0] = pltpu.async_copy(
        item_emb.at[idbuf.at[pl.ds(0, _CH)]], bufs[0], sems_in[0])
    for c in range(_NCH):
        cur = c % 2
        nxt = 1 - cur
        if c + 1 < _NCH:
            if h_out[nxt] is not None:
                h_out[nxt].wait()
            h_in[nxt] = pltpu.async_copy(
                item_emb.at[idbuf.at[pl.ds((c + 1) * _CH, _CH)]],
                bufs[nxt], sems_in[nxt])
        h_in[cur].wait()
        h_out[cur] = pltpu.async_copy(
            bufs[cur], out_pref.at[pl.ds(base_r + c * _CH, _CH)],
            sems_out[cur])
    h_out[(_NCH - 1) % 2].wait()

    ou.wait()
    op.wait()
    on.wait()


@jax.jit
def _sc_gather(user_ids, pos_ids, neg_ids, pref_ids_flat,
               user_emb, item_emb):
    mesh = plsc.VectorSubcoreMesh(core_axis_name="c", subcore_axis_name="s")
    f = pl.kernel(
        _sc_gather_body,
        compiler_params=pltpu.CompilerParams(use_tc_tiling_on_sc=False),
        out_type=[
            jax.ShapeDtypeStruct((_B * _L, _D), jnp.float32),
            jax.ShapeDtypeStruct((_B, _D), jnp.float32),
            jax.ShapeDtypeStruct((_B, _D), jnp.float32),
            jax.ShapeDtypeStruct((_B, _D), jnp.float32),
        ],
        mesh=mesh,
        scratch_types=[
            pltpu.VMEM((_BW * _L,), jnp.int32),
            pltpu.VMEM((_BW,), jnp.int32),
            pltpu.VMEM((_BW,), jnp.int32),
            pltpu.VMEM((_BW,), jnp.int32),
            pltpu.VMEM((_CH, _D), jnp.float32),
            pltpu.VMEM((_CH, _D), jnp.float32),
            pltpu.VMEM((_BW, _D), jnp.float32),
            pltpu.VMEM((_BW, _D), jnp.float32),
            pltpu.VMEM((_BW, _D), jnp.float32),
            pltpu.SemaphoreType.DMA,
            pltpu.SemaphoreType.DMA,
            pltpu.SemaphoreType.DMA,
            pltpu.SemaphoreType.DMA,
            pltpu.SemaphoreType.DMA,
            pltpu.SemaphoreType.DMA,
        ],
    )
    return f(user_ids, pos_ids, neg_ids, pref_ids_flat, user_emb, item_emb)


def _tc_att_body(pref_ref, u_ref, p_ref, n_ref, vm_ref, lm_ref,
                 dpos_ref, dneg_ref):
    pref_t = pref_ref[...].reshape(_BB, _L, _D).transpose(1, 2, 0)
    u_t = u_ref[...].transpose(1, 0)
    p_t = p_ref[...].transpose(1, 0)
    n_t = n_ref[...].transpose(1, 0)
    vm = vm_ref[...]
    lm = lm_ref[...]

    def att_pool(t_t):
        w = jnp.sum(pref_t * t_t[None, :, :], axis=1)
        e = jnp.exp(w * vm) * lm
        s = jnp.sum(e, axis=0, keepdims=True)
        en = e * vm
        av = jnp.sum(pref_t * en[:, None, :], axis=0)
        return av / s

    u_pos = u_t + att_pool(p_t)
    u_neg = u_t + att_pool(n_t)
    dpos_ref[...] = jnp.sum(jnp.square(u_pos - p_t), axis=0)
    dneg_ref[...] = jnp.sum(jnp.square(u_neg - n_t), axis=0)


@jax.jit
def _tc_att(pref_rows, u_rows, p_rows, n_rows, vm_t, lm_t):
    return pl.pallas_call(
        _tc_att_body,
        grid=(_GRID,),
        in_specs=[
            pl.BlockSpec((_BB * _L, _D), lambda i: (i, 0)),
            pl.BlockSpec((_BB, _D), lambda i: (i, 0)),
            pl.BlockSpec((_BB, _D), lambda i: (i, 0)),
            pl.BlockSpec((_BB, _D), lambda i: (i, 0)),
            pl.BlockSpec((_L, _BB), lambda i: (0, i)),
            pl.BlockSpec((_L, _BB), lambda i: (0, i)),
        ],
        out_specs=[
            pl.BlockSpec((_BB,), lambda i: (i,)),
            pl.BlockSpec((_BB,), lambda i: (i,)),
        ],
        out_shape=[
            jax.ShapeDtypeStruct((_B,), jnp.float32),
            jax.ShapeDtypeStruct((_B,), jnp.float32),
        ],
    )(pref_rows, u_rows, p_rows, n_rows, vm_t, lm_t)


def kernel(user_ids, pos_ids, neg_ids, pref_ids, n_prefs,
           user_embeddings, item_embeddings):
    pref_flat = pref_ids.reshape(-1)
    pref_rows, u_rows, p_rows, n_rows = _sc_gather(
        user_ids, pos_ids, neg_ids, pref_flat,
        user_embeddings, item_embeddings)
    vm_t = (pref_ids < _N_ITEMS).astype(jnp.float32).T
    lm_t = (jnp.arange(_L, dtype=jnp.int32)[:, None]
            < (n_prefs + 1)[None, :]).astype(jnp.float32)
    dpos, dneg = _tc_att(pref_rows, u_rows, p_rows, n_rows, vm_t, lm_t)
    return (dpos, dneg)


# speedup vs baseline: 1.9667x; 1.3122x over previous
import functools

import jax
import jax.numpy as jnp
from jax import lax
from jax.experimental import pallas as pl
from jax.experimental.pallas import tpu as pltpu
from jax.experimental.pallas import tpu_sc as plsc

_N_ITEMS = 1_000_000
_D = 32
_L = 50
_B = 16384
_LANES = 16
_NC = 2
_NS = 16
_NW = _NC * _NS
_BW = _B // _NW
_CH = 512
_NCH = (_BW * _L) // _CH

_BB = 128
_GRID = _B // _BB


def _sc_gather_body(pos_ids, neg_ids, pref_ids_flat,
                    item_emb,
                    out_pref, out_p, out_n,
                    idbuf, pid_v, nid_v,
                    rows0, rows1, prows, nrows,
                    sem_in0, sem_in1, sem_out0, sem_out1,
                    sem_upn_in, sem_upn_out):
    wid = lax.axis_index("s") * _NC + lax.axis_index("c")
    base_b = wid * _BW
    base_r = base_b * _L

    pltpu.sync_copy(pref_ids_flat.at[pl.ds(base_r, _BW * _L)], idbuf)
    pltpu.sync_copy(pos_ids.at[pl.ds(base_b, _BW)], pid_v)
    pltpu.sync_copy(neg_ids.at[pl.ds(base_b, _BW)], nid_v)

    def clamp_body(i, _):
        off = i * _LANES
        v = idbuf[pl.ds(off, _LANES)]
        idbuf[pl.ds(off, _LANES)] = jnp.minimum(
            v, jnp.full((_LANES,), _N_ITEMS - 1, jnp.int32))
        return 0
    lax.fori_loop(0, (_BW * _L) // _LANES, clamp_body, 0)

    hp = pltpu.async_copy(item_emb.at[pid_v], prows, sem_upn_in)
    hn = pltpu.async_copy(item_emb.at[nid_v], nrows, sem_upn_in)
    hp.wait()
    hn.wait()
    op = pltpu.async_copy(prows, out_p.at[pl.ds(base_b, _BW)], sem_upn_out)
    on = pltpu.async_copy(nrows, out_n.at[pl.ds(base_b, _BW)], sem_upn_out)

    bufs = (rows0, rows1)
    sems_in = (sem_in0, sem_in1)
    sems_out = (sem_out0, sem_out1)
    h_in = [None, None]
    h_out = [None, None]
    h_in[0] = pltpu.async_copy(
        item_emb.at[idbuf.at[pl.ds(0, _CH)]], bufs[0], sems_in[0])
    for c in range(_NCH):
        cur = c % 2
        nxt = 1 - cur
        if c + 1 < _NCH:
            if h_out[nxt] is not None:
                h_out[nxt].wait()
            h_in[nxt] = pltpu.async_copy(
                item_emb.at[idbuf.at[pl.ds((c + 1) * _CH, _CH)]],
                bufs[nxt], sems_in[nxt])
        h_in[cur].wait()
        h_out[cur] = pltpu.async_copy(
            bufs[cur], out_pref.at[pl.ds(base_r + c * _CH, _CH)],
            sems_out[cur])
    h_out[(_NCH - 1) % 2].wait()

    op.wait()
    on.wait()


@jax.jit
def _sc_gather(pos_ids, neg_ids, pref_ids_flat, item_emb):
    mesh = plsc.VectorSubcoreMesh(core_axis_name="c", subcore_axis_name="s")
    f = pl.kernel(
        _sc_gather_body,
        compiler_params=pltpu.CompilerParams(use_tc_tiling_on_sc=False),
        out_type=[
            jax.ShapeDtypeStruct((_B * _L, _D), jnp.float32),
            jax.ShapeDtypeStruct((_B, _D), jnp.float32),
            jax.ShapeDtypeStruct((_B, _D), jnp.float32),
        ],
        mesh=mesh,
        scratch_types=[
            pltpu.VMEM((_BW * _L,), jnp.int32),
            pltpu.VMEM((_BW,), jnp.int32),
            pltpu.VMEM((_BW,), jnp.int32),
            pltpu.VMEM((_CH, _D), jnp.float32),
            pltpu.VMEM((_CH, _D), jnp.float32),
            pltpu.VMEM((_BW, _D), jnp.float32),
            pltpu.VMEM((_BW, _D), jnp.float32),
            pltpu.SemaphoreType.DMA,
            pltpu.SemaphoreType.DMA,
            pltpu.SemaphoreType.DMA,
            pltpu.SemaphoreType.DMA,
            pltpu.SemaphoreType.DMA,
            pltpu.SemaphoreType.DMA,
        ],
    )
    return f(pos_ids, neg_ids, pref_ids_flat, item_emb)


def _tc_att_body(pref_ref, u_ref, p_ref, n_ref, vm_ref, lm_ref,
                 dpos_ref, dneg_ref):
    pref_t = pref_ref[...].reshape(_BB, _L, _D).transpose(1, 2, 0)
    u_t = u_ref[...].transpose(1, 0)
    p_t = p_ref[...].transpose(1, 0)
    n_t = n_ref[...].transpose(1, 0)
    vm = vm_ref[...]
    lm = lm_ref[...]

    def att_pool(t_t):
        w = jnp.sum(pref_t * t_t[None, :, :], axis=1)
        e = jnp.exp(w * vm) * lm
        s = jnp.sum(e, axis=0, keepdims=True)
        en = e * vm
        av = jnp.sum(pref_t * en[:, None, :], axis=0)
        return av / s

    u_pos = u_t + att_pool(p_t)
    u_neg = u_t + att_pool(n_t)
    dpos_ref[...] = jnp.sum(jnp.square(u_pos - p_t), axis=0)
    dneg_ref[...] = jnp.sum(jnp.square(u_neg - n_t), axis=0)


@jax.jit
def _tc_att(pref_rows, u_rows, p_rows, n_rows, vm_t, lm_t):
    return pl.pallas_call(
        _tc_att_body,
        grid=(_GRID,),
        in_specs=[
            pl.BlockSpec((_BB * _L, _D), lambda i: (i, 0)),
            pl.BlockSpec((_BB, _D), lambda i: (i, 0)),
            pl.BlockSpec((_BB, _D), lambda i: (i, 0)),
            pl.BlockSpec((_BB, _D), lambda i: (i, 0)),
            pl.BlockSpec((_L, _BB), lambda i: (0, i)),
            pl.BlockSpec((_L, _BB), lambda i: (0, i)),
        ],
        out_specs=[
            pl.BlockSpec((_BB,), lambda i: (i,)),
            pl.BlockSpec((_BB,), lambda i: (i,)),
        ],
        out_shape=[
            jax.ShapeDtypeStruct((_B,), jnp.float32),
            jax.ShapeDtypeStruct((_B,), jnp.float32),
        ],
    )(pref_rows, u_rows, p_rows, n_rows, vm_t, lm_t)


def kernel(user_ids, pos_ids, neg_ids, pref_ids, n_prefs,
           user_embeddings, item_embeddings):
    pref_flat = pref_ids.reshape(-1)
    pref_rows, p_rows, n_rows = _sc_gather(
        pos_ids, neg_ids, pref_flat, item_embeddings)
    u_rows = jnp.take(user_embeddings, user_ids, axis=0)
    vm_t = (pref_ids < _N_ITEMS).astype(jnp.float32).T
    lm_t = (jnp.arange(_L, dtype=jnp.int32)[:, None]
            < (n_prefs + 1)[None, :]).astype(jnp.float32)
    dpos, dneg = _tc_att(pref_rows, u_rows, p_rows, n_rows, vm_t, lm_t)
    return (dpos, dneg)


# speedup vs baseline: 3.0890x; 1.5707x over previous
import functools

import jax
import jax.numpy as jnp
from jax import lax
from jax.experimental import pallas as pl
from jax.experimental.pallas import tpu as pltpu
from jax.experimental.pallas import tpu_sc as plsc

_N_ITEMS = 1_000_000
_D = 32
_L = 50
_B = 16384
_LANES = 16
_NC = 2
_NS = 16
_NW = _NC * _NS
_BW = _B // _NW
_CH = 512
_NCH = (_BW * _L) // _CH

_BB = 128
_GRID = _B // _BB


def _sc_gather_body(pos_ids, neg_ids, pref_ids_flat,
                    item_emb,
                    out_pref, out_p, out_n,
                    idbuf, pid_v, nid_v,
                    rows0, rows1, prows, nrows,
                    sem_in0, sem_in1, sem_out0, sem_out1,
                    sem_upn_in, sem_upn_out):
    wid = lax.axis_index("s") * _NC + lax.axis_index("c")
    base_b = wid * _BW
    base_r = base_b * _L

    pltpu.sync_copy(pref_ids_flat.at[pl.ds(base_r, _BW * _L)], idbuf)
    pltpu.sync_copy(pos_ids.at[pl.ds(base_b, _BW)], pid_v)
    pltpu.sync_copy(neg_ids.at[pl.ds(base_b, _BW)], nid_v)

    def clamp_body(i, _):
        off = i * _LANES
        v = idbuf[pl.ds(off, _LANES)]
        idbuf[pl.ds(off, _LANES)] = jnp.minimum(
            v, jnp.full((_LANES,), _N_ITEMS - 1, jnp.int32))
        return 0
    lax.fori_loop(0, (_BW * _L) // _LANES, clamp_body, 0)

    hp = pltpu.async_copy(item_emb.at[pid_v], prows, sem_upn_in)
    hn = pltpu.async_copy(item_emb.at[nid_v], nrows, sem_upn_in)
    hp.wait()
    hn.wait()
    op = pltpu.async_copy(prows, out_p.at[pl.ds(base_b, _BW)], sem_upn_out)
    on = pltpu.async_copy(nrows, out_n.at[pl.ds(base_b, _BW)], sem_upn_out)

    bufs = (rows0, rows1)
    sems_in = (sem_in0, sem_in1)
    sems_out = (sem_out0, sem_out1)
    h_in = [None, None]
    h_out = [None, None]
    h_in[0] = pltpu.async_copy(
        item_emb.at[idbuf.at[pl.ds(0, _CH)]], bufs[0], sems_in[0])
    for c in range(_NCH):
        cur = c % 2
        nxt = 1 - cur
        if c + 1 < _NCH:
            if h_out[nxt] is not None:
                h_out[nxt].wait()
            h_in[nxt] = pltpu.async_copy(
                item_emb.at[idbuf.at[pl.ds((c + 1) * _CH, _CH)]],
                bufs[nxt], sems_in[nxt])
        h_in[cur].wait()
        h_out[cur] = pltpu.async_copy(
            bufs[cur], out_pref.at[pl.ds(base_r + c * _CH, _CH)],
            sems_out[cur])
    h_out[(_NCH - 1) % 2].wait()

    op.wait()
    on.wait()


@jax.jit
def _sc_gather(pos_ids, neg_ids, pref_ids_flat, item_emb):
    mesh = plsc.VectorSubcoreMesh(core_axis_name="c", subcore_axis_name="s")
    f = pl.kernel(
        _sc_gather_body,
        compiler_params=pltpu.CompilerParams(use_tc_tiling_on_sc=False),
        out_type=[
            jax.ShapeDtypeStruct((_B * _L, _D), jnp.float32),
            jax.ShapeDtypeStruct((_B, _D), jnp.float32),
            jax.ShapeDtypeStruct((_B, _D), jnp.float32),
        ],
        mesh=mesh,
        scratch_types=[
            pltpu.VMEM((_BW * _L,), jnp.int32),
            pltpu.VMEM((_BW,), jnp.int32),
            pltpu.VMEM((_BW,), jnp.int32),
            pltpu.VMEM((_CH, _D), jnp.float32),
            pltpu.VMEM((_CH, _D), jnp.float32),
            pltpu.VMEM((_BW, _D), jnp.float32),
            pltpu.VMEM((_BW, _D), jnp.float32),
            pltpu.SemaphoreType.DMA,
            pltpu.SemaphoreType.DMA,
            pltpu.SemaphoreType.DMA,
            pltpu.SemaphoreType.DMA,
            pltpu.SemaphoreType.DMA,
            pltpu.SemaphoreType.DMA,
        ],
    )
    return f(pos_ids, neg_ids, pref_ids_flat, item_emb)


_BG = _BB // 4


def _tc_att_body(pref_ref, u_ref, p_ref, n_ref, vmp_ref, np_ref,
                 dpos_ref, dneg_ref):
    pref3 = pref_ref[...]
    u3 = u_ref[...]
    p3 = p_ref[...]
    n3 = n_ref[...]

    li = lax.broadcasted_iota(jnp.int32, (128, 128), 0) // _D
    lj = lax.broadcasted_iota(jnp.int32, (128, 128), 1) // _D
    seg = (li == lj).astype(jnp.float32)

    v32 = jnp.broadcast_to(vmp_ref[0][:, :, None], (_L, _BG, 128))
    shift = (lax.broadcasted_iota(jnp.int32, (_L, _BG, 128), 2) // _D) * 8
    vm3 = ((v32 >> shift) & 1).astype(jnp.float32)

    np3 = jnp.broadcast_to(np_ref[...][None], (_L, _BG, 128))
    l3 = lax.broadcasted_iota(jnp.int32, (_L, _BG, 128), 0)
    lm3 = (l3.astype(jnp.float32) < np3 + 1.0).astype(jnp.float32)

    pv = (pref3 * vm3).reshape(_L * _BG, 128)
    lm2 = lm3.reshape(_L * _BG, 128)
    vm2 = vm3.reshape(_L * _BG, 128)
    pf2 = pref3.reshape(_L * _BG, 128)

    def att_pool(t3):
        tb = jnp.broadcast_to(t3[None], (_L, _BG, 128)).reshape(_L * _BG, 128)
        w = jnp.dot(pv * tb, seg, preferred_element_type=jnp.float32)
        e = jnp.exp(w) * lm2
        s = e.reshape(_L, _BG, 128).sum(axis=0)
        en = e * vm2
        av = (pf2 * en).reshape(_L, _BG, 128).sum(axis=0)
        return av / s

    dp = jnp.square(u3 + att_pool(p3) - p3)
    dn = jnp.square(u3 + att_pool(n3) - n3)
    dpos_ref[...] = jnp.dot(dp, seg, preferred_element_type=jnp.float32)
    dneg_ref[...] = jnp.dot(dn, seg, preferred_element_type=jnp.float32)


@jax.jit
def _tc_att(pref_g, u_g, p_g, n_g, vm_pack, np_g):
    return pl.pallas_call(
        _tc_att_body,
        grid=(_GRID,),
        in_specs=[
            pl.BlockSpec((_L, _BG, 128), lambda i: (0, i, 0)),
            pl.BlockSpec((_BG, 128), lambda i: (i, 0)),
            pl.BlockSpec((_BG, 128), lambda i: (i, 0)),
            pl.BlockSpec((_BG, 128), lambda i: (i, 0)),
            pl.BlockSpec((1, _L, _BG), lambda i: (i, 0, 0)),
            pl.BlockSpec((_BG, 128), lambda i: (i, 0)),
        ],
        out_specs=[
            pl.BlockSpec((_BG, 128), lambda i: (i, 0)),
            pl.BlockSpec((_BG, 128), lambda i: (i, 0)),
        ],
        out_shape=[
            jax.ShapeDtypeStruct((_B // 4, 128), jnp.float32),
            jax.ShapeDtypeStruct((_B // 4, 128), jnp.float32),
        ],
    )(pref_g, u_g, p_g, n_g, vm_pack, np_g)


def kernel(user_ids, pos_ids, neg_ids, pref_ids, n_prefs,
           user_embeddings, item_embeddings):
    pref_flat = pref_ids.T.reshape(-1)
    pref_rows, p_rows, n_rows = _sc_gather(
        pos_ids, neg_ids, pref_flat, item_embeddings)
    pref_g = pref_rows.reshape(_L, _B // 4, 128)
    p_g = p_rows.reshape(_B // 4, 128)
    n_g = n_rows.reshape(_B // 4, 128)
    u_g = jnp.take(user_embeddings, user_ids, axis=0).reshape(_B // 4, 128)
    vmi = (pref_ids < _N_ITEMS).astype(jnp.int32).T.reshape(_L, _B // 4, 4)
    vm_pack = (vmi[..., 0] | (vmi[..., 1] << 8)
               | (vmi[..., 2] << 16) | (vmi[..., 3] << 24))
    vm_pack = vm_pack.reshape(_L, _GRID, _BG).transpose(1, 0, 2)
    np_g = jnp.broadcast_to(
        n_prefs.astype(jnp.float32).reshape(_B // 4, 4, 1),
        (_B // 4, 4, _D)).reshape(_B // 4, 128)
    dpos_g, dneg_g = _tc_att(pref_g, u_g, p_g, n_g, vm_pack, np_g)
    dpos = dpos_g.reshape(_B // 4, 4, _D)[:, :, 0].reshape(_B)
    dneg = dneg_g.reshape(_B // 4, 4, _D)[:, :, 0].reshape(_B)
    return (dpos, dneg)
